# baseline (speedup 1.0000x reference)
import functools

import jax
import jax.numpy as jnp
from jax import lax
from jax.experimental import pallas as pl
from jax.experimental.pallas import tpu as pltpu
from jax.experimental.pallas import tpu_sc as plsc

NUM_CORES = 2
NUM_SUBCORES = 16
NW = NUM_CORES * NUM_SUBCORES
CHUNK = 400
NBUF = 4


def _build(B, V, D):
    assert B % NW == 0
    pw = B // NW
    assert pw % CHUNK == 0
    nchunks = pw // CHUNK
    assert nchunks % NBUF == 0 and nchunks >= 2 * NBUF

    mesh = plsc.VectorSubcoreMesh(
        core_axis_name="c", subcore_axis_name="s",
        num_cores=NUM_CORES, num_subcores=NUM_SUBCORES)

    @functools.partial(
        pl.kernel,
        out_type=jax.ShapeDtypeStruct((B, D), jnp.float32),
        mesh=mesh,
        scratch_types=[
            pltpu.VMEM((pw,), jnp.int32),
            pltpu.VMEM((NBUF, CHUNK, D), jnp.float32),
        ] + [pltpu.SemaphoreType.DMA] * (2 * NBUF),
        compiler_params=pltpu.CompilerParams(use_tc_tiling_on_sc=False),
    )
    def emb(weight_hbm, idx_hbm, out_hbm, idx_v, rows_v, *sems):
        gsem = sems[:NBUF]
        osem = sems[NBUF:]
        wid = lax.axis_index("s") * NUM_CORES + lax.axis_index("c")
        base = wid * pw
        pltpu.sync_copy(idx_hbm.at[pl.ds(base, pw)], idx_v)

        def gather(g, b):
            pltpu.async_copy(
                weight_hbm.at[idx_v.at[pl.ds(g * CHUNK, CHUNK)]],
                rows_v.at[b], gsem[b])

        def wait_gather(g, b):
            pltpu.make_async_copy(
                weight_hbm.at[idx_v.at[pl.ds(g * CHUNK, CHUNK)]],
                rows_v.at[b], gsem[b]).wait()

        def out(g, b):
            pltpu.async_copy(
                rows_v.at[b], out_hbm.at[pl.ds(base + g * CHUNK, CHUNK)],
                osem[b])

        def wait_out(g, b):
            pltpu.make_async_copy(
                rows_v.at[b], out_hbm.at[pl.ds(base + g * CHUNK, CHUNK)],
                osem[b]).wait()

        gather(0, 0)
        gather(1, 1)
        wait_gather(0, 0); out(0, 0); gather(2, 2)
        wait_gather(1, 1); out(1, 1); gather(3, 3)
        wait_out(0, 0); gather(4, 0); wait_gather(2, 2); out(2, 2)
        wait_out(1, 1); gather(5, 1); wait_gather(3, 3); out(3, 3)

        def step(gp, _):
            for b in range(NBUF):
                g = gp * NBUF + b
                bw = (b + 2) % NBUF
                wait_out(g - 2, bw)
                gather(g + 2, bw)
                wait_gather(g, b)
                out(g, b)
            return _

        lax.fori_loop(1, nchunks // NBUF - 1, step, 0, unroll=False)

        gl = nchunks - NBUF
        wait_out(gl - 2, 2); gather(gl + 2, 2)
        wait_out(gl - 1, 3); gather(gl + 3, 3)
        wait_gather(gl, 0); out(gl, 0)
        wait_gather(gl + 1, 1); out(gl + 1, 1)
        wait_gather(gl + 2, 2); out(gl + 2, 2)
        wait_gather(gl + 3, 3); out(gl + 3, 3)
        for b in range(NBUF):
            wait_out(gl + b, b)

    return emb


def kernel(token_ids, weight):
    B = token_ids.size
    V, D = weight.shape
    idx = jnp.reshape(token_ids, (B,)).astype(jnp.int32)
    out = _build(B, V, D)(weight, idx)
    return jnp.reshape(out, token_ids.shape + (D,))
